# Initial kernel scaffold; baseline (speedup 1.0000x reference)
import jax
import jax.numpy as jnp
from jax.experimental import pallas as pl


def kernel(from_ix, to_ix, target_paths, edge_weight_logits, edge_adjacency_logits, default_distance):
    raise NotImplementedError("write your pallas kernel here")


# speedup vs baseline: 1.0993x; 1.0993x over previous
import functools

import jax
import jax.numpy as jnp
from jax import lax
from jax.experimental import pallas as pl
from jax.experimental.pallas import tpu as pltpu
from jax.experimental.pallas import tpu_sc as plsc

_N_EDGE_ROWS = 1600001
_TOTAL = 4096 * 4 * 16
_LANES = 128
_ROWS = _TOTAL // _LANES
_NW = 32
_CPW = _ROWS // _NW
_BLK = 8
_GROUP = 16


def _sc_gather(paths2d, w_tab, a_tab):
    mesh = plsc.VectorSubcoreMesh(core_axis_name="c", subcore_axis_name="s")

    @functools.partial(
        pl.kernel,
        out_type=[
            jax.ShapeDtypeStruct((_ROWS, _LANES), jnp.float32),
            jax.ShapeDtypeStruct((_ROWS, _LANES), jnp.float32),
        ],
        mesh=mesh,
        scratch_types=[
            pltpu.VMEM((_CPW, _LANES), jnp.int32),
            pltpu.VMEM((_CPW, _LANES), jnp.float32),
            pltpu.VMEM((_CPW, _LANES), jnp.float32),
            pltpu.SemaphoreType.DMA,
            pltpu.SemaphoreType.DMA,
        ],
    )
    def k(paths_hbm, w_hbm, a_hbm, out_w_hbm, out_a_hbm,
          idx_v, w_v, a_v, w_sem, a_sem):
        wid = lax.axis_index("s") * 2 + lax.axis_index("c")
        row0 = wid * _CPW
        pltpu.sync_copy(paths_hbm.at[pl.ds(row0, _CPW)], idx_v)

        def block(b, carry):
            handles = []
            for j in range(_BLK):
                c = b * _BLK + j
                handles.append(
                    pltpu.async_copy(w_hbm.at[idx_v.at[c]], w_v.at[c], w_sem))
                handles.append(
                    pltpu.async_copy(a_hbm.at[idx_v.at[c]], a_v.at[c], a_sem))
            for h in handles:
                h.wait()
            return carry

        lax.fori_loop(0, _CPW // _BLK, block, 0)
        pltpu.sync_copy(w_v, out_w_hbm.at[pl.ds(row0, _CPW)])
        pltpu.sync_copy(a_v, out_a_hbm.at[pl.ds(row0, _CPW)])

    return k(paths2d, w_tab, a_tab)


def _tc_math(w_vals, a_vals, paths2d, default_distance):
    def body(w_ref, a_ref, p_ref, dd_ref, td_ref, lp_ref, fnd_ref):
        w = w_ref[...]
        a = a_ref[...]
        mf = (p_ref[...] != 0).astype(jnp.float32)
        sp = jnp.maximum(w, 0.0) + jnp.log(1.0 + jnp.exp(-jnp.abs(w)))
        ls = jnp.minimum(a, 0.0) - jnp.log(1.0 + jnp.exp(-jnp.abs(a)))
        li = lax.broadcasted_iota(jnp.int32, (_LANES, _LANES // _GROUP), 0)
        gi = lax.broadcasted_iota(jnp.int32, (_LANES, _LANES // _GROUP), 1)
        seg = (li // _GROUP == gi).astype(jnp.float32)
        td = jnp.dot(sp * mf, seg, preferred_element_type=jnp.float32)
        lp = jnp.dot(ls * mf, seg, preferred_element_type=jnp.float32)
        cnt = jnp.dot(mf, seg, preferred_element_type=jnp.float32)
        fnd = cnt > 0.0
        td = jnp.where(fnd, td, dd_ref[0, 0])
        td_ref[...] = td
        lp_ref[...] = lp
        fnd_ref[...] = fnd.astype(jnp.int32)

    ncols = _LANES // _GROUP
    return pl.pallas_call(
        body,
        out_shape=[
            jax.ShapeDtypeStruct((_ROWS, ncols), jnp.float32),
            jax.ShapeDtypeStruct((_ROWS, ncols), jnp.float32),
            jax.ShapeDtypeStruct((_ROWS, ncols), jnp.int32),
        ],
        in_specs=[
            pl.BlockSpec(memory_space=pltpu.VMEM),
            pl.BlockSpec(memory_space=pltpu.VMEM),
            pl.BlockSpec(memory_space=pltpu.VMEM),
            pl.BlockSpec(memory_space=pltpu.SMEM),
        ],
    )(w_vals, a_vals, paths2d, default_distance)


def kernel(from_ix, to_ix, target_paths, edge_weight_logits,
           edge_adjacency_logits, default_distance):
    paths2d = target_paths.reshape(_ROWS, _LANES)
    w_tab = edge_weight_logits.reshape(_N_EDGE_ROWS)
    a_tab = edge_adjacency_logits.reshape(_N_EDGE_ROWS)
    w_vals, a_vals = _sc_gather(paths2d, w_tab, a_tab)
    td, lp, fnd = _tc_math(w_vals, a_vals, paths2d, default_distance)
    shape = target_paths.shape[:-1]
    return (td.reshape(shape), lp.reshape(shape),
            fnd.reshape(shape).astype(jnp.bool_))


# speedup vs baseline: 2.5369x; 2.3077x over previous
import functools

import jax
import jax.numpy as jnp
from jax import lax
from jax.experimental import pallas as pl
from jax.experimental.pallas import tpu as pltpu
from jax.experimental.pallas import tpu_sc as plsc

_N_EDGE_ROWS = 1600001
_PAD_ROWS = 1601536
_TOTAL = 4096 * 4 * 16
_LANES = 128
_ROWS = _TOTAL // _LANES
_NW = 32
_CPW = _ROWS // _NW
_BLK = 8


def _sc_gather(paths2d, w_tab, a_tab):
    mesh = plsc.VectorSubcoreMesh(core_axis_name="c", subcore_axis_name="s")

    @functools.partial(
        pl.kernel,
        out_type=[
            jax.ShapeDtypeStruct((_ROWS, _LANES), jnp.float32),
            jax.ShapeDtypeStruct((_ROWS, _LANES), jnp.float32),
        ],
        mesh=mesh,
        scratch_types=[
            pltpu.VMEM((_CPW, _LANES), jnp.int32),
            pltpu.VMEM((_CPW, _LANES), jnp.float32),
            pltpu.VMEM((_CPW, _LANES), jnp.float32),
            pltpu.SemaphoreType.DMA,
            pltpu.SemaphoreType.DMA,
        ],
    )
    def k(paths_hbm, w_hbm, a_hbm, out_w_hbm, out_a_hbm,
          idx_v, w_v, a_v, w_sem, a_sem):
        wid = lax.axis_index("s") * 2 + lax.axis_index("c")
        row0 = wid * _CPW
        pltpu.sync_copy(paths_hbm.at[pl.ds(row0, _CPW)], idx_v)

        def block(b, carry):
            handles = []
            for j in range(_BLK):
                c = b * _BLK + j
                handles.append(
                    pltpu.async_copy(w_hbm.at[idx_v.at[c]], w_v.at[c], w_sem))
                handles.append(
                    pltpu.async_copy(a_hbm.at[idx_v.at[c]], a_v.at[c], a_sem))
            for h in handles:
                h.wait()
            return carry

        lax.fori_loop(0, _CPW // _BLK, block, 0)
        pltpu.sync_copy(w_v, out_w_hbm.at[pl.ds(row0, _CPW)])
        pltpu.sync_copy(a_v, out_a_hbm.at[pl.ds(row0, _CPW)])

    return k(paths2d, w_tab, a_tab)


def _tc_math(w_vals, a_vals, paths2d, default_distance, last_w, last_a):
    def body(w_ref, a_ref, p_ref, dd_ref, lw_ref, la_ref,
             td_ref, lp_ref, fnd_ref):
        p = p_ref[...]
        is_last = p == (_N_EDGE_ROWS - 1)
        w = jnp.where(is_last, lw_ref[0, 0], w_ref[...])
        a = jnp.where(is_last, la_ref[0, 0], a_ref[...])
        mf = (p != 0).astype(jnp.float32)
        sp = jnp.maximum(w, 0.0) + jnp.log(1.0 + jnp.exp(-jnp.abs(w)))
        ls = jnp.minimum(a, 0.0) - jnp.log(1.0 + jnp.exp(-jnp.abs(a)))
        li = lax.broadcasted_iota(jnp.int32, (_LANES, 8), 0)
        gi = lax.broadcasted_iota(jnp.int32, (_LANES, 8), 1)
        seg = (li // 16 == gi).astype(jnp.float32)
        td = jnp.dot(sp * mf, seg, preferred_element_type=jnp.float32)
        lp = jnp.dot(ls * mf, seg, preferred_element_type=jnp.float32)
        cnt = jnp.dot(mf, seg, preferred_element_type=jnp.float32)
        fnd = cnt > 0.0
        td = jnp.where(fnd, td, dd_ref[0, 0])
        td_ref[...] = td
        lp_ref[...] = lp
        fnd_ref[...] = fnd.astype(jnp.int32)

    return pl.pallas_call(
        body,
        out_shape=[
            jax.ShapeDtypeStruct((_ROWS, 8), jnp.float32),
            jax.ShapeDtypeStruct((_ROWS, 8), jnp.float32),
            jax.ShapeDtypeStruct((_ROWS, 8), jnp.int32),
        ],
        in_specs=[
            pl.BlockSpec(memory_space=pltpu.VMEM),
            pl.BlockSpec(memory_space=pltpu.VMEM),
            pl.BlockSpec(memory_space=pltpu.VMEM),
            pl.BlockSpec(memory_space=pltpu.SMEM),
            pl.BlockSpec(memory_space=pltpu.SMEM),
            pl.BlockSpec(memory_space=pltpu.SMEM),
        ],
    )(w_vals, a_vals, paths2d, default_distance, last_w, last_a)


def _flatten_table(tab2d):
    ext = jnp.concatenate(
        [tab2d, jnp.zeros((_PAD_ROWS - _N_EDGE_ROWS, 1), jnp.float32)],
        axis=0)
    return ext.reshape(_PAD_ROWS)


def kernel(from_ix, to_ix, target_paths, edge_weight_logits,
           edge_adjacency_logits, default_distance):
    paths2d = target_paths.reshape(_ROWS, _LANES)
    w_tab = _flatten_table(edge_weight_logits)
    a_tab = _flatten_table(edge_adjacency_logits)
    last_w = edge_weight_logits[_N_EDGE_ROWS - 1:, :]
    last_a = edge_adjacency_logits[_N_EDGE_ROWS - 1:, :]
    w_vals, a_vals = _sc_gather(paths2d, w_tab, a_tab)
    td, lp, fnd = _tc_math(w_vals, a_vals, paths2d, default_distance,
                           last_w, last_a)
    shape = target_paths.shape[:-1]
    return (td.reshape(shape), lp.reshape(shape),
            fnd.reshape(shape).astype(jnp.bool_))


# speedup vs baseline: 3.2335x; 1.2746x over previous
import functools

import jax
import jax.numpy as jnp
from jax import lax
from jax.experimental import pallas as pl
from jax.experimental.pallas import tpu as pltpu
from jax.experimental.pallas import tpu_sc as plsc

_N_EDGE_ROWS = 1600001
_PAD_ROWS = 1601536
_TOTAL = 4096 * 4 * 16
_LANES = 128
_ROWS = _TOTAL // _LANES
_NW = 32
_CPW = _ROWS // _NW
_BLK = 8


def _sc_gather(paths2d, w_tab, a_tab):
    mesh = plsc.VectorSubcoreMesh(core_axis_name="c", subcore_axis_name="s")

    @functools.partial(
        pl.kernel,
        out_type=[
            jax.ShapeDtypeStruct((_ROWS, _LANES), jnp.float32),
            jax.ShapeDtypeStruct((_ROWS, _LANES), jnp.float32),
        ],
        mesh=mesh,
        scratch_types=[
            pltpu.VMEM((_CPW, _LANES), jnp.int32),
            pltpu.VMEM((_CPW, _LANES), jnp.float32),
            pltpu.VMEM((_CPW, _LANES), jnp.float32),
            pltpu.SemaphoreType.DMA,
            pltpu.SemaphoreType.DMA,
        ],
    )
    def k(paths_hbm, w_hbm, a_hbm, out_w_hbm, out_a_hbm,
          idx_v, w_v, a_v, w_sem, a_sem):
        wid = lax.axis_index("s") * 2 + lax.axis_index("c")
        row0 = wid * _CPW
        pltpu.sync_copy(paths_hbm.at[pl.ds(row0, _CPW)], idx_v)

        def block(b, carry):
            handles = []
            for j in range(_BLK):
                c = b * _BLK + j
                handles.append(
                    pltpu.async_copy(w_hbm.at[idx_v.at[c]], w_v.at[c], w_sem))
                handles.append(
                    pltpu.async_copy(a_hbm.at[idx_v.at[c]], a_v.at[c], a_sem))
            for h in handles:
                h.wait()
            return carry

        lax.fori_loop(0, _CPW // _BLK, block, 0)
        pltpu.sync_copy(w_v, out_w_hbm.at[pl.ds(row0, _CPW)])
        pltpu.sync_copy(a_v, out_a_hbm.at[pl.ds(row0, _CPW)])

    return k(paths2d, w_tab, a_tab)


def _tc_math(w_vals, a_vals, paths2d, default_distance, last_w, last_a):
    def body(w_ref, a_ref, p_ref, dd_ref, lw_ref, la_ref,
             td_ref, lp_ref, fnd_ref):
        p = p_ref[...]
        is_last = p == (_N_EDGE_ROWS - 1)
        w = jnp.where(is_last, lw_ref[0, 0], w_ref[...])
        a = jnp.where(is_last, la_ref[0, 0], a_ref[...])
        mf = (p != 0).astype(jnp.float32)
        sp = (jnp.maximum(w, 0.0) + jnp.log(1.0 + jnp.exp(-jnp.abs(w)))) * mf
        ls = (jnp.minimum(a, 0.0) - jnp.log(1.0 + jnp.exp(-jnp.abs(a)))) * mf
        sp5 = sp.reshape(4, 2, 32, 8, _LANES)
        ls5 = ls.reshape(4, 2, 32, 8, _LANES)
        mf5 = mf.reshape(4, 2, 32, 8, _LANES)
        td = jnp.zeros((4, 32, _LANES), jnp.float32)
        lp = jnp.zeros((4, 32, _LANES), jnp.float32)
        cnt = jnp.zeros((4, 32, _LANES), jnp.float32)
        for jt in range(2):
            for ji in range(8):
                td = td + sp5[:, jt, :, ji, :]
                lp = lp + ls5[:, jt, :, ji, :]
                cnt = cnt + mf5[:, jt, :, ji, :]
        fnd = cnt > 0.0
        td = jnp.where(fnd, td, dd_ref[0, 0])
        td_ref[...] = td
        lp_ref[...] = lp
        fnd_ref[...] = fnd.astype(jnp.int32)

    return pl.pallas_call(
        body,
        out_shape=[
            jax.ShapeDtypeStruct((4, 32, _LANES), jnp.float32),
            jax.ShapeDtypeStruct((4, 32, _LANES), jnp.float32),
            jax.ShapeDtypeStruct((4, 32, _LANES), jnp.int32),
        ],
        in_specs=[
            pl.BlockSpec(memory_space=pltpu.VMEM),
            pl.BlockSpec(memory_space=pltpu.VMEM),
            pl.BlockSpec(memory_space=pltpu.VMEM),
            pl.BlockSpec(memory_space=pltpu.SMEM),
            pl.BlockSpec(memory_space=pltpu.SMEM),
            pl.BlockSpec(memory_space=pltpu.SMEM),
        ],
    )(w_vals, a_vals, paths2d, default_distance, last_w, last_a)


def _flatten_table(tab2d):
    ext = jnp.concatenate(
        [tab2d, jnp.zeros((_PAD_ROWS - _N_EDGE_ROWS, 1), jnp.float32)],
        axis=0)
    return ext.reshape(_PAD_ROWS)


def kernel(from_ix, to_ix, target_paths, edge_weight_logits,
           edge_adjacency_logits, default_distance):
    paths2d = (target_paths.reshape(32, 128, 4, 2, 8)
               .transpose(2, 3, 0, 4, 1)
               .reshape(_ROWS, _LANES))
    w_tab = _flatten_table(edge_weight_logits)
    a_tab = _flatten_table(edge_adjacency_logits)
    last_w = edge_weight_logits[_N_EDGE_ROWS - 1:, :]
    last_a = edge_adjacency_logits[_N_EDGE_ROWS - 1:, :]
    w_vals, a_vals = _sc_gather(paths2d, w_tab, a_tab)
    td, lp, fnd = _tc_math(w_vals, a_vals, paths2d, default_distance,
                           last_w, last_a)
    shape = target_paths.shape[:-1]
    td = td.transpose(1, 2, 0).reshape(shape)
    lp = lp.transpose(1, 2, 0).reshape(shape)
    fnd = fnd.transpose(1, 2, 0).reshape(shape)
    return td, lp, fnd.astype(jnp.bool_)
